# baseline (speedup 1.0000x reference)
import jax
import jax.numpy as jnp
from jax import lax
from jax.experimental import pallas as pl
from jax.experimental.pallas import tpu as pltpu
from functools import partial

_G = 8


def _leaky(x):
    return jnp.where(x >= 0, x, 0.2 * x)


def _elu(x):
    return jnp.where(x > 0, x, jnp.exp(jnp.minimum(x, 0.0)) - 1.0)


def _pair_swap(v):
    even = (lax.broadcasted_iota(jnp.int32, v.shape, 0) & 1) == 0
    return jnp.where(even, jnp.roll(v, -1, axis=0), jnp.roll(v, 1, axis=0))


def _pair_attn(a_self_src, a_self_dst, a_part_src):
    e_self = _leaky(a_self_src + a_self_dst)
    e_part = _leaky(a_part_src + a_self_dst)
    m = jnp.maximum(e_self, e_part)
    ex_s = jnp.exp(e_self - m)
    ex_p = jnp.exp(e_part - m)
    s = ex_s + ex_p + 1e-16
    return ex_s / s, ex_p / s


def _fused_kernel(L, x_ref, codes_ref, rows_ref, seg_ref, segt_ref, segi_ref,
                  w1_ref, as1_ref, ad1_ref, b1_ref,
                  w2_ref, as2_ref, ad2_ref, b2_ref, e4_ref, e4t_ref,
                  ones_ref, out_ref, idx_ref):
    x = x_ref[...]
    Gb, GL, D = x.shape
    mm = partial(jnp.dot, preferred_element_type=jnp.float32)

    m2 = (codes_ref[...] != 0).astype(jnp.float32)
    m = jnp.concatenate(
        [mm(rows_ref[s], m2) for s in range(_G)], axis=1)
    cnt = mm(m, segt_ref[...])
    cntb = mm(cnt, seg_ref[...])
    wS = m / jnp.maximum(cntb, 1.0)
    wL = (1.0 - m) / jnp.maximum(L - cntb, 1.0)
    segi = segi_ref[...]
    par = (lax.broadcasted_iota(jnp.int32, (2 * _G, GL), 0) & 1) == 0
    pieces = []
    for g in range(Gb):
        wSg = jnp.broadcast_to(wS[g][None, :], (2 * _G, GL))
        wLg = jnp.broadcast_to(wL[g][None, :], (2 * _G, GL))
        Mw = segi * jnp.where(par, wSg, wLg)
        pieces.append(mm(Mw, x[g]))
    init = jnp.concatenate(pieces, axis=0)

    w1 = w1_ref[...]
    e4 = e4_ref[...]
    e4t = e4t_ref[...]
    heads = e4.shape[0]
    af_s1 = jnp.concatenate(
        [as1_ref[hh:hh + 1, :] for hh in range(heads)], axis=1)
    af_d1 = jnp.concatenate(
        [ad1_ref[hh:hh + 1, :] for hh in range(heads)], axis=1)

    h = mm(init, w1)
    hp = _pair_swap(h)
    a_self_s = mm(h * af_s1, e4t)
    a_self_d = mm(h * af_d1, e4t)
    a_part_s = mm(hp * af_s1, e4t)
    al_self, al_part = _pair_attn(a_self_s, a_self_d, a_part_s)
    out1 = _elu(mm(al_self, e4) * h + mm(al_part, e4) * hp + b1_ref[...])

    ones_col = ones_ref[...]
    af_s2 = as2_ref[...]
    af_d2 = ad2_ref[...]
    h2 = mm(out1, w2_ref[...])
    h2p = _pair_swap(h2)
    a2_self_s = mm(h2 * af_s2, ones_col)
    a2_self_d = mm(h2 * af_d2, ones_col)
    a2_part_s = mm(h2p * af_s2, ones_col)
    a2_self, a2_part = _pair_attn(a2_self_s, a2_self_d, a2_part_s)
    out_ref[...] = a2_self * h2 + a2_part * h2p + b2_ref[...]
    nm = idx_ref.shape[1]
    idx_ref[...] = lax.broadcasted_iota(jnp.int32, (1, nm), 1) >> 1


def kernel(rna_node_features, rna_batch_idx, rna_dot_bracket_codes,
           W1, a_src1, a_dst1, b1, W2, a_src2, a_dst2, b2):
    B, L = rna_dot_bracket_codes.shape
    D = rna_node_features.shape[1]
    heads, out1 = a_src1.shape
    nslab = B // _G
    x_s = rna_node_features.reshape(nslab, _G * L, D)

    eyeh = jnp.eye(heads, dtype=jnp.float32)
    E4 = jnp.repeat(eyeh, out1, axis=1)
    E4T = jnp.repeat(eyeh, out1, axis=0)
    ONES = jnp.ones((D, 1), dtype=jnp.float32)

    SEG = jnp.repeat(jnp.eye(_G, dtype=jnp.float32), L, axis=1)
    SEGI = jnp.repeat(SEG, 2, axis=0)

    Gb = 25
    grid = (nslab // Gb,)
    riota = jnp.arange(Gb * _G)
    ROWS = (riota[None, None, :] ==
            (_G * jnp.arange(Gb)[None, :, None] + jnp.arange(_G)[:, None, None])
            ).astype(jnp.float32)

    out = pl.pallas_call(
        partial(_fused_kernel, L),
        grid=grid,
        in_specs=[
            pl.BlockSpec((Gb, _G * L, D), lambda i: (i, 0, 0)),
            pl.BlockSpec((Gb * _G, L), lambda i: (i, 0)),
            pl.BlockSpec((_G, Gb, Gb * _G), lambda i: (0, 0, 0)),
            pl.BlockSpec((_G, _G * L), lambda i: (0, 0)),
            pl.BlockSpec((_G * L, _G), lambda i: (0, 0)),
            pl.BlockSpec((2 * _G, _G * L), lambda i: (0, 0)),
            pl.BlockSpec((D, D), lambda i: (0, 0)),
            pl.BlockSpec((heads, out1), lambda i: (0, 0)),
            pl.BlockSpec((heads, out1), lambda i: (0, 0)),
            pl.BlockSpec((1, D), lambda i: (0, 0)),
            pl.BlockSpec((D, D), lambda i: (0, 0)),
            pl.BlockSpec((1, D), lambda i: (0, 0)),
            pl.BlockSpec((1, D), lambda i: (0, 0)),
            pl.BlockSpec((1, D), lambda i: (0, 0)),
            pl.BlockSpec((heads, D), lambda i: (0, 0)),
            pl.BlockSpec((D, heads), lambda i: (0, 0)),
            pl.BlockSpec((D, 1), lambda i: (0, 0)),
        ],
        out_specs=[
            pl.BlockSpec((2 * Gb * _G, D), lambda i: (i, 0)),
            pl.BlockSpec((1, 2 * B), lambda i: (0, 0)),
        ],
        out_shape=[
            jax.ShapeDtypeStruct((2 * B, D), jnp.float32),
            jax.ShapeDtypeStruct((1, 2 * B), jnp.int32),
        ],
        compiler_params=pltpu.CompilerParams(
            dimension_semantics=("arbitrary",)),
    )(x_s, rna_dot_bracket_codes, ROWS, SEG, SEG.T, SEGI, W1, a_src1, a_dst1,
      b1.reshape(1, D), W2, a_src2, a_dst2, b2.reshape(1, D), E4, E4T, ONES)

    return (out[0], out[1].reshape(2 * B))


# speedup vs baseline: 1.0676x; 1.0676x over previous
import jax
import jax.numpy as jnp
from jax import lax
from jax.experimental import pallas as pl
from jax.experimental.pallas import tpu as pltpu
from functools import partial

_G = 8


def _leaky(x):
    return jnp.where(x >= 0, x, 0.2 * x)


def _elu(x):
    return jnp.where(x > 0, x, jnp.exp(jnp.minimum(x, 0.0)) - 1.0)


def _pair_swap(v):
    even = (lax.broadcasted_iota(jnp.int32, v.shape, 0) & 1) == 0
    return jnp.where(even, jnp.roll(v, -1, axis=0), jnp.roll(v, 1, axis=0))


def _pair_attn(a_self_src, a_self_dst, a_part_src):
    e_self = _leaky(a_self_src + a_self_dst)
    e_part = _leaky(a_part_src + a_self_dst)
    m = jnp.maximum(e_self, e_part)
    ex_s = jnp.exp(e_self - m)
    ex_p = jnp.exp(e_part - m)
    s = ex_s + ex_p + 1e-16
    return ex_s / s, ex_p / s


def _fused_kernel(L, x_ref, codes_ref, seg_ref, segt_ref, segi_ref,
                  w1_ref, as1_ref, ad1_ref, b1_ref,
                  w2_ref, as2_ref, ad2_ref, b2_ref, e4_ref, e4t_ref,
                  ones_ref, out_ref, idx_ref):
    x = x_ref[...]
    Gb, GL, D = x.shape
    mm = partial(jnp.dot, preferred_element_type=jnp.float32)

    m = (codes_ref[0] != 0).astype(jnp.float32)
    cnt = mm(m, segt_ref[...])
    cntb = mm(cnt, seg_ref[...])
    wS = m / jnp.maximum(cntb, 1.0)
    wL = (1.0 - m) / jnp.maximum(L - cntb, 1.0)
    segi = segi_ref[...]
    par = (lax.broadcasted_iota(jnp.int32, (2 * _G, GL), 0) & 1) == 0
    pieces = []
    for g in range(Gb):
        wSg = jnp.broadcast_to(wS[g][None, :], (2 * _G, GL))
        wLg = jnp.broadcast_to(wL[g][None, :], (2 * _G, GL))
        Mw = segi * jnp.where(par, wSg, wLg)
        pieces.append(mm(Mw, x[g]))
    init = jnp.concatenate(pieces, axis=0)

    w1 = w1_ref[...]
    e4 = e4_ref[...]
    e4t = e4t_ref[...]
    heads = e4.shape[0]
    af_s1 = jnp.concatenate(
        [as1_ref[hh:hh + 1, :] for hh in range(heads)], axis=1)
    af_d1 = jnp.concatenate(
        [ad1_ref[hh:hh + 1, :] for hh in range(heads)], axis=1)

    h = mm(init, w1)
    hp = _pair_swap(h)
    a_self_s = mm(h * af_s1, e4t)
    a_self_d = mm(h * af_d1, e4t)
    a_part_s = mm(hp * af_s1, e4t)
    al_self, al_part = _pair_attn(a_self_s, a_self_d, a_part_s)
    out1 = _elu(mm(al_self, e4) * h + mm(al_part, e4) * hp + b1_ref[...])

    ones_col = ones_ref[...]
    af_s2 = as2_ref[...]
    af_d2 = ad2_ref[...]
    h2 = mm(out1, w2_ref[...])
    h2p = _pair_swap(h2)
    a2_self_s = mm(h2 * af_s2, ones_col)
    a2_self_d = mm(h2 * af_d2, ones_col)
    a2_part_s = mm(h2p * af_s2, ones_col)
    a2_self, a2_part = _pair_attn(a2_self_s, a2_self_d, a2_part_s)
    out_ref[...] = a2_self * h2 + a2_part * h2p + b2_ref[...]
    nm = idx_ref.shape[1]
    idx_ref[...] = lax.broadcasted_iota(jnp.int32, (1, nm), 1) >> 1


def kernel(rna_node_features, rna_batch_idx, rna_dot_bracket_codes,
           W1, a_src1, a_dst1, b1, W2, a_src2, a_dst2, b2):
    B, L = rna_dot_bracket_codes.shape
    D = rna_node_features.shape[1]
    heads, out1 = a_src1.shape
    nslab = B // _G
    x_s = rna_node_features.reshape(nslab, _G * L, D)

    eyeh = jnp.eye(heads, dtype=jnp.float32)
    E4 = jnp.repeat(eyeh, out1, axis=1)
    E4T = jnp.repeat(eyeh, out1, axis=0)
    ONES = jnp.ones((D, 1), dtype=jnp.float32)

    SEG = jnp.repeat(jnp.eye(_G, dtype=jnp.float32), L, axis=1)
    SEGI = jnp.repeat(SEG, 2, axis=0)

    Gb = 25
    grid = (nslab // Gb,)
    codes_g = rna_dot_bracket_codes.reshape(nslab // Gb, Gb, _G * L)

    out = pl.pallas_call(
        partial(_fused_kernel, L),
        grid=grid,
        in_specs=[
            pl.BlockSpec((Gb, _G * L, D), lambda i: (i, 0, 0)),
            pl.BlockSpec((1, Gb, _G * L), lambda i: (i, 0, 0)),
            pl.BlockSpec((_G, _G * L), lambda i: (0, 0)),
            pl.BlockSpec((_G * L, _G), lambda i: (0, 0)),
            pl.BlockSpec((2 * _G, _G * L), lambda i: (0, 0)),
            pl.BlockSpec((D, D), lambda i: (0, 0)),
            pl.BlockSpec((heads, out1), lambda i: (0, 0)),
            pl.BlockSpec((heads, out1), lambda i: (0, 0)),
            pl.BlockSpec((1, D), lambda i: (0, 0)),
            pl.BlockSpec((D, D), lambda i: (0, 0)),
            pl.BlockSpec((1, D), lambda i: (0, 0)),
            pl.BlockSpec((1, D), lambda i: (0, 0)),
            pl.BlockSpec((1, D), lambda i: (0, 0)),
            pl.BlockSpec((heads, D), lambda i: (0, 0)),
            pl.BlockSpec((D, heads), lambda i: (0, 0)),
            pl.BlockSpec((D, 1), lambda i: (0, 0)),
        ],
        out_specs=[
            pl.BlockSpec((2 * Gb * _G, D), lambda i: (i, 0)),
            pl.BlockSpec((1, 2 * B), lambda i: (0, 0)),
        ],
        out_shape=[
            jax.ShapeDtypeStruct((2 * B, D), jnp.float32),
            jax.ShapeDtypeStruct((1, 2 * B), jnp.int32),
        ],
        compiler_params=pltpu.CompilerParams(
            dimension_semantics=("arbitrary",)),
    )(x_s, codes_g, SEG, SEG.T, SEGI, W1, a_src1, a_dst1,
      b1.reshape(1, D), W2, a_src2, a_dst2, b2.reshape(1, D), E4, E4T, ONES)

    return (out[0], out[1].reshape(2 * B))
